# Initial kernel scaffold; baseline (speedup 1.0000x reference)
import functools

import jax
import jax.numpy as jnp
from jax import lax
from jax.experimental import pallas as pl
from jax.experimental.pallas import tpu as pltpu
from jax.experimental.pallas import tpu_sc as plsc

_TABLE_ROWS = 3375
_H = 16
_N2 = 512 * 512
_NW = 32
_PER_W = _N2 // _NW
_CHUNK = 2048
_NCHUNK = _PER_W // _CHUNK


def _bias_body(table_hbm, idx_hbm, out_hbm, table_v, idx_v, outT_v):
    wid = lax.axis_index("s") * 2 + lax.axis_index("c")
    base = wid * _PER_W

    pltpu.sync_copy(table_hbm, table_v)

    for c in range(_NCHUNK):
        start = base + c * _CHUNK
        pltpu.sync_copy(idx_hbm.at[pl.ds(start, _CHUNK)], idx_v)

        def body(g, carry):
            vidx = idx_v[pl.ds(g * 16, 16)]
            for h in range(_H):
                col = jnp.full((16,), h, jnp.int32)
                v = plsc.load_gather(table_v, [vidx, col])
                outT_v[h, pl.ds(g * 16, 16)] = v
            return carry

        lax.fori_loop(0, _CHUNK // 16, body, 0)

        for h in range(_H):
            pltpu.sync_copy(outT_v.at[h],
                            out_hbm.at[h, pl.ds(start, _CHUNK)])


@functools.partial(
    pl.kernel,
    mesh=plsc.VectorSubcoreMesh(core_axis_name="c", subcore_axis_name="s"),
    out_type=jax.ShapeDtypeStruct((_H, _N2), jnp.float32),
    scratch_types=[
        pltpu.VMEM((_TABLE_ROWS, _H), jnp.float32),
        pltpu.VMEM((_CHUNK,), jnp.int32),
        pltpu.VMEM((_H, _CHUNK), jnp.float32),
    ],
)
def _bias_call(table_hbm, idx_hbm, out_hbm, table_v, idx_v, outT_v):
    _bias_body(table_hbm, idx_hbm, out_hbm, table_v, idx_v, outT_v)


def kernel(relative_position_bias_table, relative_position_index):
    idx_flat = relative_position_index.reshape(-1)
    out = _bias_call(relative_position_bias_table, idx_flat)
    return out.reshape(1, _H, 512, 512)


# speedup vs baseline: 7.4125x; 7.4125x over previous
import functools

import jax
import jax.numpy as jnp
from jax import lax
from jax.experimental import pallas as pl
from jax.experimental.pallas import tpu as pltpu
from jax.experimental.pallas import tpu_sc as plsc

_TABLE_ROWS = 3375
_H = 16
_N2 = 512 * 512
_NW = 32
_PER_W = _N2 // _NW
_CHUNK = 2048
_NCHUNK = _PER_W // _CHUNK


def _bias_body(table_hbm, idx_hbm, out_hbm, table_v, idx_v, outT_v):
    wid = lax.axis_index("s") * 2 + lax.axis_index("c")
    base = wid * _PER_W

    pltpu.sync_copy(table_hbm, table_v)

    for c in range(_NCHUNK):
        start = base + c * _CHUNK
        pltpu.sync_copy(idx_hbm.at[pl.ds(start, _CHUNK)], idx_v)

        def body(g, carry):
            vidx = idx_v[pl.ds(g * 16, 16)] * _H
            for h in range(_H):
                v = plsc.load_gather(table_v, [vidx + h])
                outT_v[h, pl.ds(g * 16, 16)] = v
            return carry

        lax.fori_loop(0, _CHUNK // 16, body, 0)

        for h in range(_H):
            pltpu.sync_copy(outT_v.at[h],
                            out_hbm.at[h, pl.ds(start, _CHUNK)])


@functools.partial(
    pl.kernel,
    mesh=plsc.VectorSubcoreMesh(core_axis_name="c", subcore_axis_name="s"),
    compiler_params=pltpu.CompilerParams(needs_layout_passes=False),
    out_type=jax.ShapeDtypeStruct((_H, _N2), jnp.float32),
    scratch_types=[
        pltpu.VMEM((_TABLE_ROWS * _H,), jnp.float32),
        pltpu.VMEM((_CHUNK,), jnp.int32),
        pltpu.VMEM((_H, _CHUNK), jnp.float32),
    ],
)
def _bias_call(table_hbm, idx_hbm, out_hbm, table_v, idx_v, outT_v):
    _bias_body(table_hbm, idx_hbm, out_hbm, table_v, idx_v, outT_v)


def kernel(relative_position_bias_table, relative_position_index):
    idx_flat = relative_position_index.reshape(-1)
    table_flat = relative_position_bias_table.reshape(-1)
    out = _bias_call(table_flat, idx_flat)
    return out.reshape(1, _H, 512, 512)


# speedup vs baseline: 16.7187x; 2.2555x over previous
import functools

import jax
import jax.numpy as jnp
from jax import lax
from jax.experimental import pallas as pl
from jax.experimental.pallas import tpu as pltpu
from jax.experimental.pallas import tpu_sc as plsc

_TABLE_ROWS = 3375
_H = 16
_N = 512
_N2 = _N * _N
_NW = 32
_PER_W = _N2 // _NW
_CHUNK = 2048
_CROWS = _CHUNK // _N
_NCHUNK = _PER_W // _CHUNK


def _bias_body(table_hbm, idx_hbm, out_hbm, table_v, idx_v, outT_v,
               idx_s0, idx_s1, out_s0, out_s1):
    wid = lax.axis_index("s") * 2 + lax.axis_index("c")
    base = wid * _PER_W
    row0 = wid * (_PER_W // _N)
    idx_sems = (idx_s0, idx_s1)
    out_sems = (out_s0, out_s1)

    def idx_copy(c):
        return pltpu.async_copy(
            idx_hbm.at[pl.ds(base + c * _CHUNK, _CHUNK)],
            idx_v.at[c % 2], idx_sems[c % 2])

    def out_copy(c):
        return pltpu.async_copy(
            outT_v.at[c % 2],
            out_hbm.at[0, :, pl.ds(row0 + c * _CROWS, _CROWS), :],
            out_sems[c % 2])

    h_idx = [idx_copy(0), idx_copy(1)]
    pltpu.sync_copy(table_hbm, table_v)

    h_out = [None] * _NCHUNK
    for c in range(_NCHUNK):
        b = c % 2
        h_idx[c].wait()
        if c >= 2:
            h_out[c - 2].wait()

        @plsc.parallel_loop(0, _N // 16)
        def _gather(g):
            for r in range(_CROWS):
                vidx = idx_v[b, pl.ds(r * _N + g * 16, 16)] * _H
                for h in range(_H):
                    v = plsc.load_gather(table_v, [vidx + h])
                    outT_v[b, h, r, pl.ds(g * 16, 16)] = v

        h_out[c] = out_copy(c)
        if c + 2 < _NCHUNK:
            h_idx.append(idx_copy(c + 2))

    h_out[_NCHUNK - 2].wait()
    h_out[_NCHUNK - 1].wait()


@functools.partial(
    pl.kernel,
    mesh=plsc.VectorSubcoreMesh(core_axis_name="c", subcore_axis_name="s"),
    compiler_params=pltpu.CompilerParams(needs_layout_passes=False),
    out_type=jax.ShapeDtypeStruct((1, _H, _N, _N), jnp.float32),
    scratch_types=[
        pltpu.VMEM((_TABLE_ROWS * _H,), jnp.float32),
        pltpu.VMEM((2, _CHUNK), jnp.int32),
        pltpu.VMEM((2, _H, _CROWS, _N), jnp.float32),
        pltpu.SemaphoreType.DMA,
        pltpu.SemaphoreType.DMA,
        pltpu.SemaphoreType.DMA,
        pltpu.SemaphoreType.DMA,
    ],
)
def _bias_call(table_hbm, idx_hbm, out_hbm, table_v, idx_v, outT_v,
               idx_s0, idx_s1, out_s0, out_s1):
    _bias_body(table_hbm, idx_hbm, out_hbm, table_v, idx_v, outT_v,
               idx_s0, idx_s1, out_s0, out_s1)


def kernel(relative_position_bias_table, relative_position_index):
    idx_flat = relative_position_index.reshape(-1)
    table_flat = relative_position_bias_table.reshape(-1)
    return _bias_call(table_flat, idx_flat)


# speedup vs baseline: 22.1120x; 1.3226x over previous
import functools

import jax
import jax.numpy as jnp
from jax import lax
from jax.experimental import pallas as pl
from jax.experimental.pallas import tpu as pltpu
from jax.experimental.pallas import tpu_sc as plsc

_TABLE_ROWS = 3375
_H = 16
_N = 512
_N2 = _N * _N
_NW = 32
_PER_W = _N2 // _NW
_CHUNK = 2048
_CROWS = _CHUNK // _N
_NCHUNK = _PER_W // _CHUNK


def _bias_body(table_hbm, idx_hbm, out_hbm, table_v, idx_v, outT_v,
               idx_s0, idx_s1, out_s0, out_s1):
    wid = lax.axis_index("s") * 2 + lax.axis_index("c")
    base = wid * _PER_W
    row0 = wid * (_PER_W // _N)
    idx_sems = (idx_s0, idx_s1)
    out_sems = (out_s0, out_s1)

    def idx_copy(c):
        return pltpu.async_copy(
            idx_hbm.at[pl.ds(base + c * _CHUNK, _CHUNK)],
            idx_v.at[c % 2], idx_sems[c % 2])

    def out_copy(c):
        return pltpu.async_copy(
            outT_v.at[c % 2],
            out_hbm.at[0, :, pl.ds(row0 + c * _CROWS, _CROWS), :],
            out_sems[c % 2])

    h_idx = [idx_copy(0), idx_copy(1)]
    pltpu.sync_copy(table_hbm, table_v)

    h_out = [None] * _NCHUNK
    for c in range(_NCHUNK):
        b = c % 2
        h_idx[c].wait()
        if c >= 2:
            h_out[c - 2].wait()

        @plsc.parallel_loop(0, _N // 16)
        def _gather(g):
            for r in range(_CROWS):
                vidx = idx_v[b, pl.ds(r * _N + g * 16, 16)]
                for h in range(_H):
                    v = plsc.load_gather(table_v, [vidx + h * _TABLE_ROWS])
                    outT_v[b, h, r, pl.ds(g * 16, 16)] = v

        h_out[c] = out_copy(c)
        if c + 2 < _NCHUNK:
            h_idx.append(idx_copy(c + 2))

    h_out[_NCHUNK - 2].wait()
    h_out[_NCHUNK - 1].wait()


@functools.partial(
    pl.kernel,
    mesh=plsc.VectorSubcoreMesh(core_axis_name="c", subcore_axis_name="s"),
    compiler_params=pltpu.CompilerParams(needs_layout_passes=False),
    out_type=jax.ShapeDtypeStruct((1, _H, _N, _N), jnp.float32),
    scratch_types=[
        pltpu.VMEM((_TABLE_ROWS * _H,), jnp.float32),
        pltpu.VMEM((2, _CHUNK), jnp.int32),
        pltpu.VMEM((2, _H, _CROWS, _N), jnp.float32),
        pltpu.SemaphoreType.DMA,
        pltpu.SemaphoreType.DMA,
        pltpu.SemaphoreType.DMA,
        pltpu.SemaphoreType.DMA,
    ],
)
def _bias_call(table_hbm, idx_hbm, out_hbm, table_v, idx_v, outT_v,
               idx_s0, idx_s1, out_s0, out_s1):
    _bias_body(table_hbm, idx_hbm, out_hbm, table_v, idx_v, outT_v,
               idx_s0, idx_s1, out_s0, out_s1)


def kernel(relative_position_bias_table, relative_position_index):
    idx_flat = relative_position_index.reshape(-1)
    table_flat = relative_position_bias_table.T.reshape(-1)
    return _bias_call(table_flat, idx_flat)


# speedup vs baseline: 25.2240x; 1.1407x over previous
import functools

import jax
import jax.numpy as jnp
from jax import lax
from jax.experimental import pallas as pl
from jax.experimental.pallas import tpu as pltpu
from jax.experimental.pallas import tpu_sc as plsc

_TABLE_ROWS = 3375
_H = 16
_N = 512
_N2 = _N * _N
_NW = 32
_PER_W = _N2 // _NW
_CHUNK = 2048
_CROWS = _CHUNK // _N
_NCHUNK = _PER_W // _CHUNK


def _bias_body(table_hbm, idx_hbm, out_hbm, table_v, idx_v, outT_v,
               idx_sem, out_sem):
    wid = lax.axis_index("s") * 2 + lax.axis_index("c")
    base = wid * _PER_W
    row0 = wid * (_PER_W // _N)

    def start_idx(c):
        b = lax.rem(c, 2)
        pltpu.async_copy(idx_hbm.at[pl.ds(base + c * _CHUNK, _CHUNK)],
                         idx_v.at[b], idx_sem.at[b])

    def wait_idx(c):
        b = lax.rem(c, 2)
        pltpu.make_async_copy(idx_hbm.at[pl.ds(0, _CHUNK)],
                              idx_v.at[b], idx_sem.at[b]).wait()

    def start_out(c):
        b = lax.rem(c, 2)
        pltpu.async_copy(outT_v.at[b],
                         out_hbm.at[0, :, pl.ds(row0 + c * _CROWS, _CROWS), :],
                         out_sem.at[b])

    def wait_out(c):
        b = lax.rem(c, 2)
        pltpu.make_async_copy(outT_v.at[b],
                              out_hbm.at[0, :, pl.ds(0, _CROWS), :],
                              out_sem.at[b]).wait()

    start_idx(0)
    start_idx(1)
    pltpu.sync_copy(table_hbm, table_v)

    def chunk_body(c, carry):
        b = lax.rem(c, 2)
        wait_idx(c)

        @pl.when(c >= 2)
        def _():
            wait_out(c - 2)

        @plsc.parallel_loop(0, _N // 16)
        def _gather(g):
            for r in range(_CROWS):
                vidx = idx_v[b, pl.ds(r * _N + g * 16, 16)]
                for h in range(_H):
                    v = plsc.load_gather(table_v, [vidx + h * _TABLE_ROWS])
                    outT_v[b, h, r, pl.ds(g * 16, 16)] = v

        start_out(c)

        @pl.when(c + 2 < _NCHUNK)
        def _():
            start_idx(c + 2)

        return carry

    lax.fori_loop(0, _NCHUNK, chunk_body, 0)
    wait_out(_NCHUNK - 2)
    wait_out(_NCHUNK - 1)


@functools.partial(
    pl.kernel,
    mesh=plsc.VectorSubcoreMesh(core_axis_name="c", subcore_axis_name="s"),
    compiler_params=pltpu.CompilerParams(needs_layout_passes=False),
    out_type=jax.ShapeDtypeStruct((1, _H, _N, _N), jnp.float32),
    scratch_types=[
        pltpu.VMEM((_TABLE_ROWS * _H,), jnp.float32),
        pltpu.VMEM((2, _CHUNK), jnp.int32),
        pltpu.VMEM((2, _H, _CROWS, _N), jnp.float32),
        pltpu.SemaphoreType.DMA((2,)),
        pltpu.SemaphoreType.DMA((2,)),
    ],
)
def _bias_call(table_hbm, idx_hbm, out_hbm, table_v, idx_v, outT_v,
               idx_sem, out_sem):
    _bias_body(table_hbm, idx_hbm, out_hbm, table_v, idx_v, outT_v,
               idx_sem, out_sem)


def kernel(relative_position_bias_table, relative_position_index):
    idx_flat = relative_position_index.reshape(-1)
    table_flat = relative_position_bias_table.T.reshape(-1)
    return _bias_call(table_flat, idx_flat)
